# baseline (speedup 1.0000x reference)
import functools

import jax
import jax.numpy as jnp
from jax import lax
from jax.experimental import pallas as pl
from jax.experimental.pallas import tpu as pltpu
from jax.experimental.pallas import tpu_sc as plsc

_N = 10000
_E = 320000
_D = 128
_NC = 2
_NS = 16
_L = 16
_NW = _NC * _NS
_EPW = _E // _NW
_CH = 80
_NCHUNK = _EPW // _CH
_NP = 10240
_RPT = _NP // _NS
_NB = 4
_CHA = 48
_CHB = _CH - _CHA

_MBLK = 2000


def _mm_body(x_ref, w_ref, b_ref, o_ref):
    o_ref[...] = (
        jnp.dot(x_ref[...], w_ref[...], preferred_element_type=jnp.float32)
        + b_ref[...]
    )


def _dense_transform(x, w, b2d):
    return pl.pallas_call(
        _mm_body,
        grid=(_N // _MBLK,),
        in_specs=[
            pl.BlockSpec((_MBLK, _D), lambda i: (i, 0)),
            pl.BlockSpec((_D, _D), lambda i: (0, 0)),
            pl.BlockSpec((1, _D), lambda i: (0, 0)),
        ],
        out_specs=pl.BlockSpec((_MBLK, _D), lambda i: (i, 0)),
        out_shape=jax.ShapeDtypeStruct((_N, _D), jnp.float32),
    )(x, w, b2d)


def _maybe(cond, fn):
    if isinstance(cond, (bool,)):
        if cond:
            fn()
    else:
        pl.when(cond)(fn)


def _sc_body(h_hbm, src_hbm, dst_hbm, adj_hbm, out_hbm,
             s0, s1, s2, s3, d0, d1, d2, d3, e0, e1, e2, e3,
             a0, a1, a2, a3, g0, g1, g2, g3,
             is0, is1, is2, is3, ds0, ds1, ds2, ds3,
             gs0, gs1, gs2, gs3, ss0, ss1, ss2, ss3,
             acc_sh):
    srcb = (s0, s1, s2, s3)
    dstA = (d0, d1, d2, d3)
    dstB = (e0, e1, e2, e3)
    adjb = (a0, a1, a2, a3)
    rows = (g0, g1, g2, g3)
    isem = (is0, is1, is2, is3)
    dsem = (ds0, ds1, ds2, ds3)
    gsem = (gs0, gs1, gs2, gs3)
    ssem = (ss0, ss1, ss2, ss3)

    cid = lax.axis_index("c")
    sid = lax.axis_index("s")
    wid = sid * _NC + cid
    ebase = wid * _EPW

    def _zrow(r, _):
        for d in range(_D // _L):
            rows[0][r, pl.ds(d * _L, _L)] = jnp.zeros((_L,), jnp.float32)
        return 0
    lax.fori_loop(0, _CH, _zrow, 0)
    for k in range(_RPT // _CH):
        pltpu.sync_copy(rows[0], acc_sh.at[pl.ds(sid * _RPT + k * _CH, _CH)])
    plsc.subcore_barrier()

    def issue_sa(c, b):
        off = ebase + c * _CH
        pltpu.async_copy(src_hbm.at[pl.ds(off, _CH)], srcb[b], isem[b])
        pltpu.async_copy(adj_hbm.at[pl.ds(off, _CH)], adjb[b], isem[b])

    def wait_sa(b):
        pltpu.make_async_copy(src_hbm.at[pl.ds(0, _CH)], srcb[b], isem[b]).wait()
        pltpu.make_async_copy(adj_hbm.at[pl.ds(0, _CH)], adjb[b], isem[b]).wait()

    def issue_dst(c, b):
        off = ebase + c * _CH
        pltpu.async_copy(dst_hbm.at[pl.ds(off, _CHA)], dstA[b], dsem[b])
        pltpu.async_copy(dst_hbm.at[pl.ds(off + _CHA, _CHB)], dstB[b], dsem[b])

    def wait_dst(b):
        pltpu.make_async_copy(dst_hbm.at[pl.ds(0, _CHA)], dstA[b], dsem[b]).wait()
        pltpu.make_async_copy(dst_hbm.at[pl.ds(0, _CHB)], dstB[b], dsem[b]).wait()

    def issue_gather(b):
        pltpu.async_copy(h_hbm.at[srcb[b]], rows[b], gsem[b])

    def wait_gather(b):
        pltpu.make_async_copy(h_hbm.at[srcb[b]], rows[b], gsem[b]).wait()

    def issue_scatter_a(b):
        pltpu.async_copy(
            rows[b].at[pl.ds(0, _CHA)], acc_sh.at[dstA[b]], ssem[b], add=True)

    def issue_scatter_b(b):
        pltpu.async_copy(
            rows[b].at[pl.ds(_CHA, _CHB)], acc_sh.at[dstB[b]], ssem[b],
            add=True)

    def wait_scatter(b):
        pltpu.make_async_copy(
            rows[b].at[pl.ds(0, _CHA)], acc_sh.at[dstA[b]], ssem[b]).wait()
        pltpu.make_async_copy(
            rows[b].at[pl.ds(_CHA, _CHB)], acc_sh.at[dstB[b]], ssem[b]).wait()

    def compute(b, g0, g1):
        def _group(g, _):
            av = adjb[b][pl.ds(g * _L, _L)]
            for i in range(_L):
                a = jnp.broadcast_to(av[i], (_L,))
                e = g * _L + i
                for d in range(_D // _L):
                    sl = pl.ds(d * _L, _L)
                    rows[b][e, sl] = rows[b][e, sl] * a
            return 0
        lax.fori_loop(g0, g1, _group, 0)

    def step(c, b):
        b2 = (b + 2) % _NB
        wait_gather(b)

        def _w():
            wait_scatter(b2)
        _maybe(c >= 2, _w)

        def _g():
            wait_sa(b2)
            issue_gather(b2)
        _maybe(c + 2 < _NCHUNK, _g)

        def _p3():
            issue_sa(c + 3, (b + 3) % _NB)
        _maybe(c + 3 < _NCHUNK, _p3)

        def _p2():
            issue_dst(c + 2, b2)
        _maybe(c + 2 < _NCHUNK, _p2)

        wait_dst(b)
        compute(b, 0, _CHA // _L)
        issue_scatter_a(b)
        compute(b, _CHA // _L, _CH // _L)
        issue_scatter_b(b)

    issue_sa(0, 0)
    issue_sa(1, 1)
    issue_sa(2, 2)
    issue_dst(0, 0)
    issue_dst(1, 1)
    wait_sa(0)
    issue_gather(0)
    wait_sa(1)
    issue_gather(1)
    step(0, 0)

    def _main(i, _):
        c0 = 1 + i * _NB
        for j in range(_NB):
            step(c0 + j, (1 + j) % _NB)
        return 0
    lax.fori_loop(0, (_NCHUNK - 1) // _NB, _main, 0)

    wait_scatter((_NCHUNK - 2) % _NB)
    wait_scatter((_NCHUNK - 1) % _NB)

    plsc.subcore_barrier()

    pltpu.sync_copy(
        acc_sh.at[pl.ds(sid * _RPT, _RPT)],
        out_hbm.at[pl.ds(cid * _NP + sid * _RPT, _RPT)],
    )


_sc_spmm = functools.partial(
    pl.kernel,
    out_type=jax.ShapeDtypeStruct((_NC * _NP, _D), jnp.float32),
    mesh=plsc.VectorSubcoreMesh(
        core_axis_name="c", subcore_axis_name="s",
        num_cores=_NC, num_subcores=_NS),
    scratch_types=(
        [pltpu.VMEM((_CH,), jnp.int32) for _ in range(_NB)]
        + [pltpu.VMEM((_CHA,), jnp.int32) for _ in range(_NB)]
        + [pltpu.VMEM((_CHB,), jnp.int32) for _ in range(_NB)]
        + [pltpu.VMEM((_CH,), jnp.float32) for _ in range(_NB)]
        + [pltpu.VMEM((_CH, _D), jnp.float32) for _ in range(_NB)]
        + [pltpu.SemaphoreType.DMA for _ in range(4 * _NB)]
        + [pltpu.VMEM_SHARED((_NP, _D), jnp.float32)]
    ),
)(_sc_body)


def _add_body(p_ref, o_ref):
    o_ref[...] = p_ref[0] + p_ref[1]


def _combine(partials):
    return pl.pallas_call(
        _add_body,
        grid=(_N // 1000,),
        in_specs=[pl.BlockSpec((2, 1000, _D), lambda i: (0, i, 0))],
        out_specs=pl.BlockSpec((1000, _D), lambda i: (i, 0)),
        out_shape=jax.ShapeDtypeStruct((_N, _D), jnp.float32),
    )(partials)


def kernel(input, edge_index, adj_values, weight, bias):
    h = _dense_transform(input, weight, bias.reshape(1, _D))
    src = edge_index[0]
    dst = edge_index[1]
    partials = _sc_spmm(h, src, dst, adj_values)
    return _combine(partials.reshape(_NC, _NP, _D))


# speedup vs baseline: 1.0093x; 1.0093x over previous
import functools

import jax
import jax.numpy as jnp
from jax import lax
from jax.experimental import pallas as pl
from jax.experimental.pallas import tpu as pltpu
from jax.experimental.pallas import tpu_sc as plsc

_N = 10000
_E = 320000
_D = 128
_NC = 2
_NS = 16
_L = 16
_NW = _NC * _NS
_EPW = _E // _NW
_CH = 80
_NCHUNK = _EPW // _CH
_NP = 10240
_RPT = _NP // _NS
_NB = 4

_MBLK = 2000


def _mm_body(x_ref, w_ref, b_ref, o_ref):
    o_ref[...] = (
        jnp.dot(x_ref[...], w_ref[...], preferred_element_type=jnp.float32)
        + b_ref[...]
    )


def _dense_transform(x, w, b2d):
    return pl.pallas_call(
        _mm_body,
        grid=(_N // _MBLK,),
        in_specs=[
            pl.BlockSpec((_MBLK, _D), lambda i: (i, 0)),
            pl.BlockSpec((_D, _D), lambda i: (0, 0)),
            pl.BlockSpec((1, _D), lambda i: (0, 0)),
        ],
        out_specs=pl.BlockSpec((_MBLK, _D), lambda i: (i, 0)),
        out_shape=jax.ShapeDtypeStruct((_N, _D), jnp.float32),
    )(x, w, b2d)


def _maybe(cond, fn):
    if isinstance(cond, (bool,)):
        if cond:
            fn()
    else:
        pl.when(cond)(fn)


def _sc_body(h_hbm, src_hbm, dst_hbm, adj_hbm, out_hbm,
             s0, s1, s2, s3, d0, d1, d2, d3, a0, a1, a2, a3,
             g0, g1, g2, g3,
             is0, is1, is2, is3, ds0, ds1, ds2, ds3,
             gs0, gs1, gs2, gs3, ss0, ss1, ss2, ss3,
             acc_sh):
    srcb = (s0, s1, s2, s3)
    dstb = (d0, d1, d2, d3)
    adjb = (a0, a1, a2, a3)
    rows = (g0, g1, g2, g3)
    isem = (is0, is1, is2, is3)
    dsem = (ds0, ds1, ds2, ds3)
    gsem = (gs0, gs1, gs2, gs3)
    ssem = (ss0, ss1, ss2, ss3)

    cid = lax.axis_index("c")
    sid = lax.axis_index("s")
    wid = sid * _NC + cid
    ebase = wid * _EPW


    def issue_sa(c, b):
        off = ebase + c * _CH
        pltpu.async_copy(src_hbm.at[pl.ds(off, _CH)], srcb[b], isem[b])
        pltpu.async_copy(adj_hbm.at[pl.ds(off, _CH)], adjb[b], isem[b])

    def wait_sa(b):
        pltpu.make_async_copy(src_hbm.at[pl.ds(0, _CH)], srcb[b], isem[b]).wait()
        pltpu.make_async_copy(adj_hbm.at[pl.ds(0, _CH)], adjb[b], isem[b]).wait()

    def issue_dst(c, b):
        off = ebase + c * _CH
        pltpu.async_copy(dst_hbm.at[pl.ds(off, _CH)], dstb[b], dsem[b])

    def wait_dst(b):
        pltpu.make_async_copy(dst_hbm.at[pl.ds(0, _CH)], dstb[b], dsem[b]).wait()

    def issue_gather(b):
        pltpu.async_copy(h_hbm.at[srcb[b]], rows[b], gsem[b])

    def wait_gather(b):
        pltpu.make_async_copy(h_hbm.at[srcb[b]], rows[b], gsem[b]).wait()

    def issue_scatter(b):
        pltpu.async_copy(rows[b], acc_sh.at[dstb[b]], ssem[b], add=True)

    def wait_scatter(b):
        pltpu.make_async_copy(rows[b], acc_sh.at[dstb[b]], ssem[b]).wait()

    def compute(b, g0, g1):
        def _group(g, _):
            av = adjb[b][pl.ds(g * _L, _L)]
            for i in range(_L):
                a = jnp.broadcast_to(av[i], (_L,))
                e = g * _L + i
                for d in range(_D // _L):
                    sl = pl.ds(d * _L, _L)
                    rows[b][e, sl] = rows[b][e, sl] * a
            return 0
        lax.fori_loop(g0, g1, _group, 0)

    def step(c, b):
        b2 = (b + 2) % _NB
        wait_gather(b)

        def _w():
            wait_scatter(b2)
        _maybe(c >= 2, _w)

        def _g():
            wait_sa(b2)
            issue_gather(b2)
        _maybe(c + 2 < _NCHUNK, _g)

        def _p3():
            issue_sa(c + 3, (b + 3) % _NB)
        _maybe(c + 3 < _NCHUNK, _p3)

        def _p2():
            issue_dst(c + 2, b2)
        _maybe(c + 2 < _NCHUNK, _p2)

        wait_dst(b)
        compute(b, 0, _CH // _L)
        issue_scatter(b)

    issue_sa(0, 0)
    issue_sa(1, 1)
    issue_sa(2, 2)
    issue_dst(0, 0)
    issue_dst(1, 1)

    def _zrow(r, _):
        for d in range(_D // _L):
            rows[0][r, pl.ds(d * _L, _L)] = jnp.zeros((_L,), jnp.float32)
        return 0
    lax.fori_loop(0, _CH, _zrow, 0)
    for k in range(_RPT // _CH):
        pltpu.async_copy(
            rows[0], acc_sh.at[pl.ds(sid * _RPT + k * _CH, _CH)], gsem[3])
    for k in range(_RPT // _CH):
        pltpu.make_async_copy(
            rows[0], acc_sh.at[pl.ds(sid * _RPT + k * _CH, _CH)],
            gsem[3]).wait()
    plsc.subcore_barrier()

    wait_sa(0)
    issue_gather(0)
    wait_sa(1)
    issue_gather(1)
    step(0, 0)

    def _main(i, _):
        c0 = 1 + i * _NB
        for j in range(_NB):
            step(c0 + j, (1 + j) % _NB)
        return 0
    lax.fori_loop(0, (_NCHUNK - 1) // _NB, _main, 0)

    wait_scatter((_NCHUNK - 2) % _NB)
    wait_scatter((_NCHUNK - 1) % _NB)

    plsc.subcore_barrier()

    pltpu.sync_copy(
        acc_sh.at[pl.ds(sid * _RPT, _RPT)],
        out_hbm.at[pl.ds(cid * _NP + sid * _RPT, _RPT)],
    )


_sc_spmm = functools.partial(
    pl.kernel,
    out_type=jax.ShapeDtypeStruct((_NC * _NP, _D), jnp.float32),
    mesh=plsc.VectorSubcoreMesh(
        core_axis_name="c", subcore_axis_name="s",
        num_cores=_NC, num_subcores=_NS),
    scratch_types=(
        [pltpu.VMEM((_CH,), jnp.int32) for _ in range(_NB)]
        + [pltpu.VMEM((_CH,), jnp.int32) for _ in range(_NB)]
        + [pltpu.VMEM((_CH,), jnp.float32) for _ in range(_NB)]
        + [pltpu.VMEM((_CH, _D), jnp.float32) for _ in range(_NB)]
        + [pltpu.SemaphoreType.DMA for _ in range(4 * _NB)]
        + [pltpu.VMEM_SHARED((_NP, _D), jnp.float32)]
    ),
)(_sc_body)


def _add_body(p_ref, o_ref):
    o_ref[...] = p_ref[0] + p_ref[1]


def _combine(partials):
    return pl.pallas_call(
        _add_body,
        grid=(_N // 1000,),
        in_specs=[pl.BlockSpec((2, 1000, _D), lambda i: (0, i, 0))],
        out_specs=pl.BlockSpec((1000, _D), lambda i: (i, 0)),
        out_shape=jax.ShapeDtypeStruct((_N, _D), jnp.float32),
    )(partials)


def kernel(input, edge_index, adj_values, weight, bias):
    h = _dense_transform(input, weight, bias.reshape(1, _D))
    src = edge_index[0]
    dst = edge_index[1]
    partials = _sc_spmm(h, src, dst, adj_values)
    return _combine(partials.reshape(_NC, _NP, _D))
